# Initial kernel scaffold; baseline (speedup 1.0000x reference)
import jax
import jax.numpy as jnp
from jax.experimental import pallas as pl


def kernel(feature, xyz, neigh_idx, W_m1, g_m1, b_m1, W_x1, g_x1, b_x1, W_fc1, W_mlp1, g_a1, b_a1, W_x2, g_x2, b_x2, W_fc2, W_mlp2, g_a2, b_a2, W_m2, g_m2, b_m2, W_sc, g_sc, b_sc):
    raise NotImplementedError("write your pallas kernel here")


# speedup vs baseline: 134.2636x; 134.2636x over previous
import functools

import jax
import jax.numpy as jnp
from jax import lax
from jax.experimental import pallas as pl
from jax.experimental.pallas import tpu as pltpu
from jax.experimental.pallas import tpu_sc as plsc

_EPS = 1e-5


def _bn(y, g, b):
    return y * (g / jnp.sqrt(1.0 + _EPS)) + b


def _mm(x, w):
    return jnp.dot(x, w, preferred_element_type=jnp.float32)


def _ka_body(P, N, fT, xyzb, nidx, W_m1, g_m1, b_m1, t1_o, gidx_o):
    i = pl.program_id(0)
    base = (i * P) // N * N
    fpc = jnp.maximum(_bn(_mm(fT[...], W_m1[...]), g_m1[...], b_m1[...]), 0.0)
    zeros5 = jnp.zeros((P, 5), jnp.float32)
    t1_o[...] = jnp.concatenate([xyzb[...], fpc, zeros5], axis=1)
    gidx_o[...] = nidx[...] + base


def _kc_body(P, K, e1, xyzb, W_x1, g_x1, b_x1, W_fc1, W_mlp1, g_a1, b_a1,
             W_x2, g_x2, b_x2, t2_o, fxyz2_o):
    E = P * K
    ed = e1[...]
    nb = ed[:, 0:3]
    fpc_nb = ed[:, 3:11]
    tile = jnp.broadcast_to(xyzb[...][:, None, :], (P, K, 3)).reshape(E, 3)
    rel = tile - nb
    dis = jnp.sqrt(jnp.sum(rel * rel, axis=1, keepdims=True))
    fx = jnp.concatenate([dis, rel, tile, nb], axis=1)
    fx1 = jnp.maximum(_bn(_mm(fx, W_x1[...]), g_x1[...], b_x1[...]), 0.0)
    fcat = jnp.concatenate([fpc_nb, fx1], axis=1)
    att = _mm(fcat, W_fc1[...]).reshape(P, K, 16)
    m = jnp.max(att, axis=1, keepdims=True)
    e = jnp.exp(att - m)
    s = jnp.sum(e, axis=1, keepdims=True)
    sc = e / s
    agg = jnp.sum(fcat.reshape(P, K, 16) * sc, axis=1)
    fagg = jnp.maximum(_bn(_mm(agg, W_mlp1[...]), g_a1[...], b_a1[...]), 0.0)
    t2_o[...] = jnp.concatenate([fagg, jnp.zeros((P, 8), jnp.float32)], axis=1)
    fxyz2_o[...] = jnp.maximum(_bn(_mm(fx1, W_x2[...]), g_x2[...], b_x2[...]), 0.0)


def _ke_body(P, K, e2, fxyz2, fT, W_fc2, W_mlp2, g_a2, b_a2,
             W_m2, g_m2, b_m2, W_sc, g_sc, b_sc, out_o):
    E = P * K
    fcat2 = jnp.concatenate([e2[...][:, 0:8], fxyz2[...]], axis=1)
    att = _mm(fcat2, W_fc2[...]).reshape(P, K, 16)
    m = jnp.max(att, axis=1, keepdims=True)
    e = jnp.exp(att - m)
    s = jnp.sum(e, axis=1, keepdims=True)
    sc = e / s
    agg = jnp.sum(fcat2.reshape(P, K, 16) * sc, axis=1)
    fagg2 = jnp.maximum(_bn(_mm(agg, W_mlp2[...]), g_a2[...], b_a2[...]), 0.0)
    f_out = _bn(_mm(fagg2, W_m2[...]), g_m2[...], b_m2[...])
    scb = _bn(_mm(fT[...], W_sc[...]), g_sc[...], b_sc[...])
    y = f_out + scb
    out_o[...] = jnp.where(y >= 0.0, y, 0.2 * y)


def _sc_gather(table, gidx, E, CH):
    info = plsc.get_sparse_core_info()
    NC, NS = info.num_cores, info.num_subcores
    NW = NC * NS
    EW = E // NW
    mesh = plsc.VectorSubcoreMesh(core_axis_name="c", subcore_axis_name="s")

    @functools.partial(
        pl.kernel,
        out_type=jax.ShapeDtypeStruct((E, 16), jnp.float32),
        mesh=mesh,
        scratch_types=[
            pltpu.VMEM((CH,), jnp.int32),
            pltpu.VMEM((CH, 16), jnp.float32),
            pltpu.SemaphoreType.DMA,
        ],
        compiler_params=pltpu.CompilerParams(use_tc_tiling_on_sc=False),
    )
    def k(table_h, gidx_h, out_h, idx_v, rows_v, sem):
        wid = lax.axis_index("s") * NC + lax.axis_index("c")
        base = wid * EW

        def body(j, carry):
            off = base + j * CH
            pltpu.sync_copy(gidx_h.at[pl.ds(off, CH)], idx_v)
            pltpu.async_copy(table_h.at[idx_v], rows_v, sem).wait()
            pltpu.sync_copy(rows_v, out_h.at[pl.ds(off, CH)])
            return carry

        lax.fori_loop(0, EW // CH, body, 0)

    return k(table, gidx)


def _full(shape):
    nd = len(shape)
    return pl.BlockSpec(shape, lambda i: (0,) * nd)


def kernel(feature, xyz, neigh_idx, W_m1, g_m1, b_m1, W_x1, g_x1, b_x1,
           W_fc1, W_mlp1, g_a1, b_a1, W_x2, g_x2, b_x2, W_fc2, W_mlp2,
           g_a2, b_a2, W_m2, g_m2, b_m2, W_sc, g_sc, b_sc):
    B, Cin, N, _ = feature.shape
    K = neigh_idx.shape[2]
    BN = B * N
    E = BN * K
    P = 1000
    G = BN // P

    fT = feature.reshape(B, Cin, N).transpose(0, 2, 1).reshape(BN, Cin)
    xyzf = xyz.reshape(BN, 3)
    nidx2 = neigh_idx.reshape(BN, K)
    r1 = lambda v: v.reshape(1, -1)
    (g_m1, b_m1, g_x1, b_x1, g_a1, b_a1, g_x2, b_x2, g_a2, b_a2,
     g_m2, b_m2, g_sc, b_sc) = map(r1, (g_m1, b_m1, g_x1, b_x1, g_a1, b_a1,
                                        g_x2, b_x2, g_a2, b_a2, g_m2, b_m2,
                                        g_sc, b_sc))

    t1, gidx2 = pl.pallas_call(
        functools.partial(_ka_body, P, N),
        grid=(G,),
        in_specs=[
            pl.BlockSpec((P, Cin), lambda i: (i, 0)),
            pl.BlockSpec((P, 3), lambda i: (i, 0)),
            pl.BlockSpec((P, K), lambda i: (i, 0)),
            _full(W_m1.shape), _full(g_m1.shape), _full(b_m1.shape),
        ],
        out_specs=[
            pl.BlockSpec((P, 16), lambda i: (i, 0)),
            pl.BlockSpec((P, K), lambda i: (i, 0)),
        ],
        out_shape=[
            jax.ShapeDtypeStruct((BN, 16), jnp.float32),
            jax.ShapeDtypeStruct((BN, K), jnp.int32),
        ],
        compiler_params=pltpu.CompilerParams(vmem_limit_bytes=128 * 1024 * 1024),
    )(fT, xyzf, nidx2, W_m1, g_m1, b_m1)

    gidx = gidx2.reshape(E)

    edges1 = _sc_gather(t1, gidx, E, 2000)

    t2, fxyz2 = pl.pallas_call(
        functools.partial(_kc_body, P, K),
        grid=(G,),
        in_specs=[
            pl.BlockSpec((P * K, 16), lambda i: (i, 0)),
            pl.BlockSpec((P, 3), lambda i: (i, 0)),
            _full(W_x1.shape), _full(g_x1.shape), _full(b_x1.shape),
            _full(W_fc1.shape), _full(W_mlp1.shape),
            _full(g_a1.shape), _full(b_a1.shape),
            _full(W_x2.shape), _full(g_x2.shape), _full(b_x2.shape),
        ],
        out_specs=[
            pl.BlockSpec((P, 16), lambda i: (i, 0)),
            pl.BlockSpec((P * K, 8), lambda i: (i, 0)),
        ],
        out_shape=[
            jax.ShapeDtypeStruct((BN, 16), jnp.float32),
            jax.ShapeDtypeStruct((E, 8), jnp.float32),
        ],
        compiler_params=pltpu.CompilerParams(vmem_limit_bytes=128 * 1024 * 1024),
    )(edges1, xyzf, W_x1, g_x1, b_x1, W_fc1, W_mlp1, g_a1, b_a1,
      W_x2, g_x2, b_x2)

    edges2 = _sc_gather(t2, gidx, E, 2000)

    out_pm = pl.pallas_call(
        functools.partial(_ke_body, P, K),
        grid=(G,),
        in_specs=[
            pl.BlockSpec((P * K, 16), lambda i: (i, 0)),
            pl.BlockSpec((P * K, 8), lambda i: (i, 0)),
            pl.BlockSpec((P, Cin), lambda i: (i, 0)),
            _full(W_fc2.shape), _full(W_mlp2.shape),
            _full(g_a2.shape), _full(b_a2.shape),
            _full(W_m2.shape), _full(g_m2.shape), _full(b_m2.shape),
            _full(W_sc.shape), _full(g_sc.shape), _full(b_sc.shape),
        ],
        out_specs=pl.BlockSpec((P, 32), lambda i: (i, 0)),
        out_shape=jax.ShapeDtypeStruct((BN, 32), jnp.float32),
        compiler_params=pltpu.CompilerParams(vmem_limit_bytes=128 * 1024 * 1024),
    )(edges2, fxyz2, fT, W_fc2, W_mlp2, g_a2, b_a2, W_m2, g_m2, b_m2,
      W_sc, g_sc, b_sc)

    return out_pm.reshape(B, N, 32).transpose(0, 2, 1)[:, :, :, None]


# speedup vs baseline: 227.5675x; 1.6949x over previous
import functools

import jax
import jax.numpy as jnp
from jax import lax
from jax.experimental import pallas as pl
from jax.experimental.pallas import tpu as pltpu
from jax.experimental.pallas import tpu_sc as plsc

_EPS = 1e-5


def _scale(g):
    return g * (1.0 / jnp.sqrt(1.0 + _EPS))


def _mm(x, w):
    return jnp.dot(x, w, preferred_element_type=jnp.float32)


def _roll(x, s):
    return jnp.roll(x, s, axis=1)


def _blockdiag(w16):
    row = jnp.concatenate([w16] * 8, axis=1)
    t = jnp.concatenate([row] * 8, axis=0)
    bi = lax.broadcasted_iota(jnp.int32, (128, 128), 0) // 16
    bj = lax.broadcasted_iota(jnp.int32, (128, 128), 1) // 16
    return jnp.where(bi == bj, t, 0.0)


def _pack_vec16(v16):
    return jnp.concatenate([v16] * 8, axis=1)


def _pad16(w, r0, c0):
    r, c = w.shape
    cols = ([jnp.zeros((r, c0), jnp.float32)] if c0 else []) + [w] + \
        ([jnp.zeros((r, 16 - c0 - c), jnp.float32)] if 16 - c0 - c else [])
    w = jnp.concatenate(cols, axis=1) if len(cols) > 1 else cols[0]
    rows = ([jnp.zeros((r0, 16), jnp.float32)] if r0 else []) + [w] + \
        ([jnp.zeros((16 - r0 - r, 16), jnp.float32)] if 16 - r0 - r else [])
    return jnp.concatenate(rows, axis=0) if len(rows) > 1 else rows[0]


def _group_softmax_weights(att, R):
    m = jnp.maximum(att, _roll(att, 16))
    m = jnp.maximum(m, _roll(m, 32))
    m = jnp.maximum(m, _roll(m, 64))
    m3 = m.reshape(R // 2, 2, 128)
    mp = jnp.max(m3, axis=1, keepdims=True)
    mfull = jnp.broadcast_to(mp, (R // 2, 2, 128)).reshape(R, 128)
    e = jnp.exp(att - mfull)
    s = e + _roll(e, 16)
    s = s + _roll(s, 32)
    s = s + _roll(s, 64)
    s3 = s.reshape(R // 2, 2, 128)
    sp = jnp.sum(s3, axis=1, keepdims=True)
    sfull = jnp.broadcast_to(sp, (R // 2, 2, 128)).reshape(R, 128)
    return e / sfull


def _group_sum(x, R):
    s = x + _roll(x, 16)
    s = s + _roll(s, 32)
    s = s + _roll(s, 64)
    s3 = s.reshape(R // 2, 2, 128)
    sp = jnp.sum(s3, axis=1, keepdims=True)
    return jnp.broadcast_to(sp, (R // 2, 2, 128)).reshape(R, 128)


def _ka_body(P, N, fT, xyzb, nidxp, W_m1, g_m1, b_m1, t1_o, gidx_o):
    i = pl.program_id(0)
    base = (i * P) // N * N
    fpc = _mm(fT[...], W_m1[...]) * _scale(g_m1[...]) + b_m1[...]
    fpc = jnp.maximum(fpc, 0.0)
    zeros5 = jnp.zeros((P, 5), jnp.float32)
    t1_o[...] = jnp.concatenate([xyzb[...], fpc, zeros5], axis=1)
    gidx_o[...] = nidxp[...] + base


def _kc_body(P, K, e1, xyzb, W_x1, g_x1, b_x1, W_fc1, W_mlp1, g_a1, b_a1,
             W_x2, g_x2, b_x2, t2_o, fxyz2_o):
    R = P * K // 8
    ed = e1[...]
    li = lax.broadcasted_iota(jnp.int32, (R, 128), 1) % 16

    x16 = jnp.concatenate([xyzb[...], jnp.zeros((P, 13), jnp.float32)], axis=1)
    x128 = jnp.concatenate([x16] * 8, axis=1)
    tile = jnp.broadcast_to(x128[:, None, :], (P, 2, 128)).reshape(R, 128)

    rel = tile - ed
    r2 = rel * rel
    d2 = r2 + _roll(r2, -1) + _roll(r2, -2)
    dis = jnp.sqrt(d2)

    fx = jnp.where(li == 0, dis, 0.0)
    fx = fx + jnp.where((li >= 1) & (li <= 3), _roll(rel, 1), 0.0)
    fx = fx + jnp.where((li >= 4) & (li <= 6), _roll(tile, 4), 0.0)
    fx = fx + jnp.where((li >= 7) & (li <= 9), _roll(ed, 7), 0.0)

    wx1 = _blockdiag(_pad16(W_x1[...], 0, 0))
    s1 = _pack_vec16(_pad16(_scale(g_x1[...]), 0, 0)[0:1])
    bb1 = _pack_vec16(_pad16(b_x1[...], 0, 0)[0:1])
    fx1 = jnp.maximum(_mm(fx, wx1) * s1 + bb1, 0.0)

    fcat = jnp.where(li < 8, _roll(ed, -3), _roll(fx1, 8))

    att = _mm(fcat, _blockdiag(W_fc1[...]))
    sc = _group_softmax_weights(att, R)
    aggf = _group_sum(fcat * sc, R)
    aggP = aggf.reshape(R // 2, 2, 128)[:, 0, 0:16]

    fagg = _mm(aggP, W_mlp1[...]) * _scale(g_a1[...]) + b_a1[...]
    fagg = jnp.maximum(fagg, 0.0)
    t2_o[...] = jnp.concatenate([fagg, jnp.zeros((P, 8), jnp.float32)], axis=1)

    wx2 = _blockdiag(_pad16(W_x2[...], 0, 8))
    s2 = _pack_vec16(_pad16(_scale(g_x2[...]), 0, 8)[0:1])
    bb2 = _pack_vec16(_pad16(b_x2[...], 0, 8)[0:1])
    fxyz2_o[...] = jnp.maximum(_mm(fx1, wx2) * s2 + bb2, 0.0)


def _ke_body(P, K, e2, fxyz2, fT, W_fc2, W_mlp2, g_a2, b_a2,
             W_m2, g_m2, b_m2, W_sc, g_sc, b_sc, out_o):
    R = P * K // 8
    li = lax.broadcasted_iota(jnp.int32, (R, 128), 1) % 16
    fcat2 = jnp.where(li < 8, e2[...], fxyz2[...])

    att = _mm(fcat2, _blockdiag(W_fc2[...]))
    sc = _group_softmax_weights(att, R)
    aggf = _group_sum(fcat2 * sc, R)
    aggP = aggf.reshape(R // 2, 2, 128)[:, 0, 0:16]

    fagg2 = _mm(aggP, W_mlp2[...]) * _scale(g_a2[...]) + b_a2[...]
    fagg2 = jnp.maximum(fagg2, 0.0)
    f_out = _mm(fagg2, W_m2[...]) * _scale(g_m2[...]) + b_m2[...]
    scb = _mm(fT[...], W_sc[...]) * _scale(g_sc[...]) + b_sc[...]
    y = f_out + scb
    out_o[...] = jnp.where(y >= 0.0, y, 0.2 * y)


def _sc_gather(table, gidx, E, CH):
    info = plsc.get_sparse_core_info()
    NC, NS = info.num_cores, info.num_subcores
    NW = NC * NS
    EW = E // NW
    mesh = plsc.VectorSubcoreMesh(core_axis_name="c", subcore_axis_name="s")

    @functools.partial(
        pl.kernel,
        out_type=jax.ShapeDtypeStruct((E, 16), jnp.float32),
        mesh=mesh,
        scratch_types=[
            pltpu.VMEM((CH,), jnp.int32),
            pltpu.VMEM((CH, 16), jnp.float32),
            pltpu.SemaphoreType.DMA,
        ],
        compiler_params=pltpu.CompilerParams(use_tc_tiling_on_sc=False),
    )
    def k(table_h, gidx_h, out_h, idx_v, rows_v, sem):
        wid = lax.axis_index("s") * NC + lax.axis_index("c")
        base = wid * EW

        def body(j, carry):
            off = base + j * CH
            pltpu.sync_copy(gidx_h.at[pl.ds(off, CH)], idx_v)
            pltpu.async_copy(table_h.at[idx_v], rows_v, sem).wait()
            pltpu.sync_copy(rows_v, out_h.at[pl.ds(off, CH)])
            return carry

        lax.fori_loop(0, EW // CH, body, 0)

    return k(table, gidx)


def _full(shape):
    nd = len(shape)
    return pl.BlockSpec(shape, lambda i: (0,) * nd)


def kernel(feature, xyz, neigh_idx, W_m1, g_m1, b_m1, W_x1, g_x1, b_x1,
           W_fc1, W_mlp1, g_a1, b_a1, W_x2, g_x2, b_x2, W_fc2, W_mlp2,
           g_a2, b_a2, W_m2, g_m2, b_m2, W_sc, g_sc, b_sc):
    B, Cin, N, _ = feature.shape
    K = neigh_idx.shape[2]
    BN = B * N
    E = BN * K
    P = 2000
    G = BN // P
    RB = P * K // 8
    RI = P * K // 128

    fT = feature.reshape(B, Cin, N).transpose(0, 2, 1).reshape(BN, Cin)
    xyzf = xyz.reshape(BN, 3)
    nidxp = neigh_idx.reshape(G, E // 128 // G, 128)
    r1 = lambda v: v.reshape(1, -1)
    (g_m1, b_m1, g_x1, b_x1, g_a1, b_a1, g_x2, b_x2, g_a2, b_a2,
     g_m2, b_m2, g_sc, b_sc) = map(r1, (g_m1, b_m1, g_x1, b_x1, g_a1, b_a1,
                                        g_x2, b_x2, g_a2, b_a2, g_m2, b_m2,
                                        g_sc, b_sc))
    vmem = pltpu.CompilerParams(vmem_limit_bytes=128 * 1024 * 1024)

    t1, gidxp = pl.pallas_call(
        functools.partial(_ka_body, P, N),
        grid=(G,),
        in_specs=[
            pl.BlockSpec((P, Cin), lambda i: (i, 0)),
            pl.BlockSpec((P, 3), lambda i: (i, 0)),
            pl.BlockSpec((1, RI, 128), lambda i: (i, 0, 0)),
            _full(W_m1.shape), _full(g_m1.shape), _full(b_m1.shape),
        ],
        out_specs=[
            pl.BlockSpec((P, 16), lambda i: (i, 0)),
            pl.BlockSpec((1, RI, 128), lambda i: (i, 0, 0)),
        ],
        out_shape=[
            jax.ShapeDtypeStruct((BN, 16), jnp.float32),
            jax.ShapeDtypeStruct((G, E // 128 // G, 128), jnp.int32),
        ],
        compiler_params=vmem,
    )(fT, xyzf, nidxp, W_m1, g_m1, b_m1)

    gidx = gidxp.reshape(E)

    edges1 = _sc_gather(t1, gidx, E, 2000).reshape(E // 8, 128)

    t2, fxyz2 = pl.pallas_call(
        functools.partial(_kc_body, P, K),
        grid=(G,),
        in_specs=[
            pl.BlockSpec((RB, 128), lambda i: (i, 0)),
            pl.BlockSpec((P, 3), lambda i: (i, 0)),
            _full(W_x1.shape), _full(g_x1.shape), _full(b_x1.shape),
            _full(W_fc1.shape), _full(W_mlp1.shape),
            _full(g_a1.shape), _full(b_a1.shape),
            _full(W_x2.shape), _full(g_x2.shape), _full(b_x2.shape),
        ],
        out_specs=[
            pl.BlockSpec((P, 16), lambda i: (i, 0)),
            pl.BlockSpec((RB, 128), lambda i: (i, 0)),
        ],
        out_shape=[
            jax.ShapeDtypeStruct((BN, 16), jnp.float32),
            jax.ShapeDtypeStruct((E // 8, 128), jnp.float32),
        ],
        compiler_params=vmem,
    )(edges1, xyzf, W_x1, g_x1, b_x1, W_fc1, W_mlp1, g_a1, b_a1,
      W_x2, g_x2, b_x2)

    edges2 = _sc_gather(t2, gidx, E, 2000).reshape(E // 8, 128)

    out_pm = pl.pallas_call(
        functools.partial(_ke_body, P, K),
        grid=(G,),
        in_specs=[
            pl.BlockSpec((RB, 128), lambda i: (i, 0)),
            pl.BlockSpec((RB, 128), lambda i: (i, 0)),
            pl.BlockSpec((P, Cin), lambda i: (i, 0)),
            _full(W_fc2.shape), _full(W_mlp2.shape),
            _full(g_a2.shape), _full(b_a2.shape),
            _full(W_m2.shape), _full(g_m2.shape), _full(b_m2.shape),
            _full(W_sc.shape), _full(g_sc.shape), _full(b_sc.shape),
        ],
        out_specs=pl.BlockSpec((P, 32), lambda i: (i, 0)),
        out_shape=jax.ShapeDtypeStruct((BN, 32), jnp.float32),
        compiler_params=vmem,
    )(edges2, fxyz2, fT, W_fc2, W_mlp2, g_a2, b_a2, W_m2, g_m2, b_m2,
      W_sc, g_sc, b_sc)

    return out_pm.reshape(B, N, 32).transpose(0, 2, 1)[:, :, :, None]


# speedup vs baseline: 236.1211x; 1.0376x over previous
import functools

import jax
import jax.numpy as jnp
from jax import lax
from jax.experimental import pallas as pl
from jax.experimental.pallas import tpu as pltpu
from jax.experimental.pallas import tpu_sc as plsc

_EPS = 1e-5


def _scale(g):
    return g * (1.0 / jnp.sqrt(1.0 + _EPS))


def _mm(x, w):
    return jnp.dot(x, w, preferred_element_type=jnp.float32)


def _roll(x, s):
    return jnp.roll(x, s, axis=1)


def _blockdiag(w16):
    row = jnp.concatenate([w16] * 8, axis=1)
    t = jnp.concatenate([row] * 8, axis=0)
    bi = lax.broadcasted_iota(jnp.int32, (128, 128), 0) // 16
    bj = lax.broadcasted_iota(jnp.int32, (128, 128), 1) // 16
    return jnp.where(bi == bj, t, 0.0)


def _pack_vec16(v16):
    return jnp.concatenate([v16] * 8, axis=1)


def _pad16(w, r0, c0):
    r, c = w.shape
    cols = ([jnp.zeros((r, c0), jnp.float32)] if c0 else []) + [w] + \
        ([jnp.zeros((r, 16 - c0 - c), jnp.float32)] if 16 - c0 - c else [])
    w = jnp.concatenate(cols, axis=1) if len(cols) > 1 else cols[0]
    rows = ([jnp.zeros((r0, 16), jnp.float32)] if r0 else []) + [w] + \
        ([jnp.zeros((16 - r0 - r, 16), jnp.float32)] if 16 - r0 - r else [])
    return jnp.concatenate(rows, axis=0) if len(rows) > 1 else rows[0]


def _group_softmax_weights(att, R):
    m = jnp.maximum(att, _roll(att, 16))
    m = jnp.maximum(m, _roll(m, 32))
    m = jnp.maximum(m, _roll(m, 64))
    m3 = m.reshape(R // 2, 2, 128)
    mp = jnp.max(m3, axis=1, keepdims=True)
    mfull = jnp.broadcast_to(mp, (R // 2, 2, 128)).reshape(R, 128)
    e = jnp.exp(att - mfull)
    s = e + _roll(e, 16)
    s = s + _roll(s, 32)
    s = s + _roll(s, 64)
    s3 = s.reshape(R // 2, 2, 128)
    sp = jnp.sum(s3, axis=1, keepdims=True)
    sfull = jnp.broadcast_to(sp, (R // 2, 2, 128)).reshape(R, 128)
    return e / sfull


def _group_sum(x, R):
    s = x + _roll(x, 16)
    s = s + _roll(s, 32)
    s = s + _roll(s, 64)
    s3 = s.reshape(R // 2, 2, 128)
    sp = jnp.sum(s3, axis=1, keepdims=True)
    return jnp.broadcast_to(sp, (R // 2, 2, 128)).reshape(R, 128)


def _ka_body(P, N, fT, xyzb, nidxp, W_m1, g_m1, b_m1, t1_o, gidx_o):
    i = pl.program_id(0)
    base = (i * P) // N * N
    fpc = _mm(fT[...], W_m1[...]) * _scale(g_m1[...]) + b_m1[...]
    fpc = jnp.maximum(fpc, 0.0)
    zeros5 = jnp.zeros((P, 5), jnp.float32)
    t1_o[...] = jnp.concatenate([xyzb[...], fpc, zeros5], axis=1)
    gidx_o[...] = nidxp[...] + base


def _kc_body(P, K, e1, xyzb, W_x1, g_x1, b_x1, W_fc1, W_mlp1, g_a1, b_a1,
             W_x2, g_x2, b_x2, t2_o, fxyz2_o):
    R = P * K // 8
    ed = e1[...]
    li = lax.broadcasted_iota(jnp.int32, (R, 128), 1) % 16

    x16 = jnp.concatenate([xyzb[...], jnp.zeros((P, 13), jnp.float32)], axis=1)
    x128 = jnp.concatenate([x16] * 8, axis=1)
    tile = jnp.broadcast_to(x128[:, None, :], (P, 2, 128)).reshape(R, 128)

    rel = tile - ed
    r2 = rel * rel
    d2 = r2 + _roll(r2, -1) + _roll(r2, -2)
    dis = jnp.sqrt(d2)

    fx = jnp.where(li == 0, dis, 0.0)
    fx = fx + jnp.where((li >= 1) & (li <= 3), _roll(rel, 1), 0.0)
    fx = fx + jnp.where((li >= 4) & (li <= 6), _roll(tile, 4), 0.0)
    fx = fx + jnp.where((li >= 7) & (li <= 9), _roll(ed, 7), 0.0)

    wx1 = _blockdiag(_pad16(W_x1[...], 0, 0))
    s1 = _pack_vec16(_pad16(_scale(g_x1[...]), 0, 0)[0:1])
    bb1 = _pack_vec16(_pad16(b_x1[...], 0, 0)[0:1])
    fx1 = jnp.maximum(_mm(fx, wx1) * s1 + bb1, 0.0)

    fcat = jnp.where(li < 8, _roll(ed, -3), _roll(fx1, 8))

    att = _mm(fcat, _blockdiag(W_fc1[...]))
    sc = _group_softmax_weights(att, R)
    aggf = _group_sum(fcat * sc, R)
    aggP = aggf.reshape(R // 2, 2, 128)[:, 0, 0:16]

    fagg = _mm(aggP, W_mlp1[...]) * _scale(g_a1[...]) + b_a1[...]
    fagg = jnp.maximum(fagg, 0.0)
    t2_o[...] = jnp.concatenate([fagg, jnp.zeros((P, 8), jnp.float32)], axis=1)

    wx2 = _blockdiag(_pad16(W_x2[...], 0, 8))
    s2 = _pack_vec16(_pad16(_scale(g_x2[...]), 0, 8)[0:1])
    bb2 = _pack_vec16(_pad16(b_x2[...], 0, 8)[0:1])
    fxyz2_o[...] = jnp.maximum(_mm(fx1, wx2) * s2 + bb2, 0.0)


def _ke_body(P, K, e2, fxyz2, fT, W_fc2, W_mlp2, g_a2, b_a2,
             W_m2, g_m2, b_m2, W_sc, g_sc, b_sc, out_o):
    R = P * K // 8
    li = lax.broadcasted_iota(jnp.int32, (R, 128), 1) % 16
    fcat2 = jnp.where(li < 8, e2[...], fxyz2[...])

    att = _mm(fcat2, _blockdiag(W_fc2[...]))
    sc = _group_softmax_weights(att, R)
    aggf = _group_sum(fcat2 * sc, R)
    aggP = aggf.reshape(R // 2, 2, 128)[:, 0, 0:16]

    fagg2 = _mm(aggP, W_mlp2[...]) * _scale(g_a2[...]) + b_a2[...]
    fagg2 = jnp.maximum(fagg2, 0.0)
    f_out = _mm(fagg2, W_m2[...]) * _scale(g_m2[...]) + b_m2[...]
    scb = _mm(fT[...], W_sc[...]) * _scale(g_sc[...]) + b_sc[...]
    y = f_out + scb
    out_o[...] = jnp.where(y >= 0.0, y, 0.2 * y)


def _sc_gather(table, gidx, E, CH):
    info = plsc.get_sparse_core_info()
    NC, NS = info.num_cores, info.num_subcores
    NW = NC * NS
    EW = E // NW
    mesh = plsc.VectorSubcoreMesh(core_axis_name="c", subcore_axis_name="s")

    NCH = EW // CH
    assert NCH % 2 == 1

    @functools.partial(
        pl.kernel,
        out_type=jax.ShapeDtypeStruct((E, 16), jnp.float32),
        mesh=mesh,
        scratch_types=[
            pltpu.VMEM((CH,), jnp.int32),
            pltpu.VMEM((CH,), jnp.int32),
            pltpu.VMEM((CH, 16), jnp.float32),
            pltpu.VMEM((CH, 16), jnp.float32),
            pltpu.SemaphoreType.DMA,
            pltpu.SemaphoreType.DMA,
        ],
        compiler_params=pltpu.CompilerParams(use_tc_tiling_on_sc=False),
    )
    def k(table_h, gidx_h, out_h, idx0, idx1, rows0, rows1, sem0, sem1):
        wid = lax.axis_index("s") * NC + lax.axis_index("c")
        base = wid * EW
        idx = (idx0, idx1)
        rows = (rows0, rows1)
        sem = (sem0, sem1)

        pltpu.sync_copy(gidx_h.at[pl.ds(base, CH)], idx0)
        pltpu.async_copy(table_h.at[idx0], rows0, sem0)

        def body(j, carry):
            for s in range(2):
                off = base + (j + s) * CH
                nxt = (s + 1) % 2

                @pl.when(j + s + 1 < NCH)
                def _():
                    pltpu.sync_copy(gidx_h.at[pl.ds(off + CH, CH)], idx[nxt])
                    pltpu.async_copy(table_h.at[idx[nxt]], rows[nxt], sem[nxt])

                pltpu.make_async_copy(table_h.at[idx[s]], rows[s], sem[s]).wait()
                pltpu.sync_copy(rows[s], out_h.at[pl.ds(off, CH)])
            return carry

        lax.fori_loop(0, (NCH - 1) // 2, lambda j, c: body(2 * j, c), 0)

        off = base + (NCH - 1) * CH
        pltpu.make_async_copy(table_h.at[idx0], rows0, sem0).wait()
        pltpu.sync_copy(rows0, out_h.at[pl.ds(off, CH)])

    return k(table, gidx)


def _full(shape):
    nd = len(shape)
    return pl.BlockSpec(shape, lambda i: (0,) * nd)


def kernel(feature, xyz, neigh_idx, W_m1, g_m1, b_m1, W_x1, g_x1, b_x1,
           W_fc1, W_mlp1, g_a1, b_a1, W_x2, g_x2, b_x2, W_fc2, W_mlp2,
           g_a2, b_a2, W_m2, g_m2, b_m2, W_sc, g_sc, b_sc):
    B, Cin, N, _ = feature.shape
    K = neigh_idx.shape[2]
    BN = B * N
    E = BN * K
    P = 2000
    G = BN // P
    RB = P * K // 8
    RI = P * K // 128

    fT = feature.reshape(B, Cin, N).transpose(0, 2, 1).reshape(BN, Cin)
    xyzf = xyz.reshape(BN, 3)
    nidxp = neigh_idx.reshape(G, E // 128 // G, 128)
    r1 = lambda v: v.reshape(1, -1)
    (g_m1, b_m1, g_x1, b_x1, g_a1, b_a1, g_x2, b_x2, g_a2, b_a2,
     g_m2, b_m2, g_sc, b_sc) = map(r1, (g_m1, b_m1, g_x1, b_x1, g_a1, b_a1,
                                        g_x2, b_x2, g_a2, b_a2, g_m2, b_m2,
                                        g_sc, b_sc))
    vmem = pltpu.CompilerParams(vmem_limit_bytes=128 * 1024 * 1024)

    t1, gidxp = pl.pallas_call(
        functools.partial(_ka_body, P, N),
        grid=(G,),
        in_specs=[
            pl.BlockSpec((P, Cin), lambda i: (i, 0)),
            pl.BlockSpec((P, 3), lambda i: (i, 0)),
            pl.BlockSpec((1, RI, 128), lambda i: (i, 0, 0)),
            _full(W_m1.shape), _full(g_m1.shape), _full(b_m1.shape),
        ],
        out_specs=[
            pl.BlockSpec((P, 16), lambda i: (i, 0)),
            pl.BlockSpec((1, RI, 128), lambda i: (i, 0, 0)),
        ],
        out_shape=[
            jax.ShapeDtypeStruct((BN, 16), jnp.float32),
            jax.ShapeDtypeStruct((G, E // 128 // G, 128), jnp.int32),
        ],
        compiler_params=vmem,
    )(fT, xyzf, nidxp, W_m1, g_m1, b_m1)

    gidx = gidxp.reshape(E)

    edges1 = _sc_gather(t1, gidx, E, 2000).reshape(E // 8, 128)

    t2, fxyz2 = pl.pallas_call(
        functools.partial(_kc_body, P, K),
        grid=(G,),
        in_specs=[
            pl.BlockSpec((RB, 128), lambda i: (i, 0)),
            pl.BlockSpec((P, 3), lambda i: (i, 0)),
            _full(W_x1.shape), _full(g_x1.shape), _full(b_x1.shape),
            _full(W_fc1.shape), _full(W_mlp1.shape),
            _full(g_a1.shape), _full(b_a1.shape),
            _full(W_x2.shape), _full(g_x2.shape), _full(b_x2.shape),
        ],
        out_specs=[
            pl.BlockSpec((P, 16), lambda i: (i, 0)),
            pl.BlockSpec((RB, 128), lambda i: (i, 0)),
        ],
        out_shape=[
            jax.ShapeDtypeStruct((BN, 16), jnp.float32),
            jax.ShapeDtypeStruct((E // 8, 128), jnp.float32),
        ],
        compiler_params=vmem,
    )(edges1, xyzf, W_x1, g_x1, b_x1, W_fc1, W_mlp1, g_a1, b_a1,
      W_x2, g_x2, b_x2)

    edges2 = _sc_gather(t2, gidx, E, 2000).reshape(E // 8, 128)

    out_pm = pl.pallas_call(
        functools.partial(_ke_body, P, K),
        grid=(G,),
        in_specs=[
            pl.BlockSpec((RB, 128), lambda i: (i, 0)),
            pl.BlockSpec((RB, 128), lambda i: (i, 0)),
            pl.BlockSpec((P, Cin), lambda i: (i, 0)),
            _full(W_fc2.shape), _full(W_mlp2.shape),
            _full(g_a2.shape), _full(b_a2.shape),
            _full(W_m2.shape), _full(g_m2.shape), _full(b_m2.shape),
            _full(W_sc.shape), _full(g_sc.shape), _full(b_sc.shape),
        ],
        out_specs=pl.BlockSpec((P, 32), lambda i: (i, 0)),
        out_shape=jax.ShapeDtypeStruct((BN, 32), jnp.float32),
        compiler_params=vmem,
    )(edges2, fxyz2, fT, W_fc2, W_mlp2, g_a2, b_a2, W_m2, g_m2, b_m2,
      W_sc, g_sc, b_sc)

    return out_pm.reshape(B, N, 32).transpose(0, 2, 1)[:, :, :, None]
